# baseline (speedup 1.0000x reference)
import functools
import jax
import jax.numpy as jnp
from jax import lax
from jax.experimental import pallas as pl
from jax.experimental.pallas import tpu as pltpu
from jax.experimental.pallas import tpu_sc as plsc


def kernel(x, person_emb):
    B, T, P, D = x.shape
    R = 10
    NCH = T // R

    info = plsc.get_sparse_core_info()
    NC, NS = info.num_cores, info.num_subcores
    assert NC * NS == B

    mesh = plsc.VectorSubcoreMesh(core_axis_name="c", subcore_axis_name="s")

    @functools.partial(
        pl.kernel,
        mesh=mesh,
        out_type=jax.ShapeDtypeStruct((B, T, P, D), person_emb.dtype),
        scratch_types=[
            pltpu.VMEM((R, P, D), person_emb.dtype),
            pltpu.SemaphoreType.DMA,
        ],
        compiler_params=pltpu.CompilerParams(use_tc_tiling_on_sc=True),
    )
    def k(emb_hbm, out_hbm, buf, sem):
        w = lax.axis_index("s") * NC + lax.axis_index("c")
        fills = [pltpu.async_copy(emb_hbm, buf.at[i], sem) for i in range(R)]
        for f in fills:
            f.wait()
        streams = [
            pltpu.async_copy(buf, out_hbm.at[w, pl.ds(j * R, R)], sem)
            for j in range(NCH)
        ]
        for s in streams:
            s.wait()

    return k(person_emb)
